# Initial kernel scaffold; baseline (speedup 1.0000x reference)
import jax
import jax.numpy as jnp
from jax.experimental import pallas as pl


def kernel(g, patterns, beta, edges, triangles):
    raise NotImplementedError("write your pallas kernel here")


# speedup vs baseline: 17.1061x; 17.1061x over previous
import functools

import jax
import jax.numpy as jnp
from jax.experimental import pallas as pl
from jax.experimental.pallas import tpu as pltpu


def _body(idx_ref, g_ref, p_ref, beta_ref, out_ref, h_ref, acc_ref,
          *, B, N, K, BLK, nblocks, num_simplices):
    i = pl.program_id(0)
    beta = beta_ref[0, 0]

    @pl.when(i == 0)
    def _init():
        for bb in range(B):
            h_ref[bb] = jnp.dot(g_ref[bb], p_ref[...],
                                preferred_element_type=jnp.float32)
        acc_ref[0] = 0.0
        acc_ref[1] = jnp.sum(g_ref[...] ** 2)

    idx = idx_ref[...]
    iota = jax.lax.broadcasted_iota(jnp.int32, (BLK, N), 1)
    a = ((idx[:, 0:1] == iota).astype(jnp.float32)
         + (idx[:, 1:2] == iota).astype(jnp.float32)
         + (idx[:, 2:3] == iota).astype(jnp.float32))
    valid = idx[:, 0:1] >= 0

    tot = jnp.float32(0.0)
    for bb in range(B):
        hs = jnp.dot(a, h_ref[bb], preferred_element_type=jnp.float32)
        s = jnp.sum(jnp.exp(beta * hs), axis=1, keepdims=True)
        tot = tot + jnp.sum(jnp.where(valid, jnp.log(s), 0.0))
    acc_ref[0] = acc_ref[0] + tot

    @pl.when(i == nblocks - 1)
    def _fin():
        energy = (-(1.0 / (beta * num_simplices)) * acc_ref[0]
                  - 2.0 * acc_ref[1]) / (B * N)
        out_ref[...] = jnp.reshape(energy, (1, 1))


def kernel(g, patterns, beta, edges, triangles):
    B, N, D = g.shape
    K = patterns.shape[1]
    m2, m3 = edges.shape[0], triangles.shape[0]
    num_simplices = m2 + m3

    edges = edges.astype(jnp.int32)
    triangles = triangles.astype(jnp.int32)
    idx_all = jnp.concatenate(
        [jnp.concatenate([edges, jnp.full((m2, 1), -1, jnp.int32)], axis=1),
         triangles], axis=0)

    BLK = 1024
    n_pad = ((num_simplices + BLK - 1) // BLK) * BLK
    idx_all = jnp.pad(idx_all, ((0, n_pad - num_simplices), (0, 0)),
                      constant_values=-1)
    nblocks = n_pad // BLK
    beta_arr = jnp.reshape(beta.astype(jnp.float32), (1, 1))

    body = functools.partial(_body, B=B, N=N, K=K, BLK=BLK, nblocks=nblocks,
                             num_simplices=num_simplices)
    out = pl.pallas_call(
        body,
        grid=(nblocks,),
        in_specs=[
            pl.BlockSpec((BLK, 3), lambda i: (i, 0)),
            pl.BlockSpec((B, N, D), lambda i: (0, 0, 0)),
            pl.BlockSpec((D, K), lambda i: (0, 0)),
            pl.BlockSpec((1, 1), lambda i: (0, 0)),
        ],
        out_specs=pl.BlockSpec((1, 1), lambda i: (0, 0)),
        out_shape=jax.ShapeDtypeStruct((1, 1), jnp.float32),
        scratch_shapes=[
            pltpu.VMEM((B, N, K), jnp.float32),
            pltpu.SMEM((2,), jnp.float32),
        ],
    )(idx_all, g, patterns, beta_arr)
    return jnp.reshape(out, ())


# speedup vs baseline: 24.7263x; 1.4455x over previous
import functools

import jax
import jax.numpy as jnp
from jax.experimental import pallas as pl
from jax.experimental.pallas import tpu as pltpu


def _onehot(idx_col, rows, n, dtype):
    iota = jax.lax.broadcasted_iota(jnp.int32, (rows, n), 1)
    return (idx_col == iota).astype(dtype)


def _body(tri_ref, edge_ref, g_ref, p_ref, beta_ref, out_ref,
          h_ref, logg_ref, c_ref, acc_ref,
          *, B, N, K, BLK, nblocks, num_simplices):
    i = pl.program_id(0)
    beta = beta_ref[0, 0]

    @pl.when(i == 0)
    def _init():
        for bb in range(B):
            hb = jnp.dot(g_ref[bb], p_ref[...],
                         preferred_element_type=jnp.float32)
            h_ref[:, bb * K:(bb + 1) * K] = hb.astype(jnp.bfloat16)
            eb = jnp.exp(beta * hb).astype(jnp.bfloat16)
            gb = jax.lax.dot_general(eb, eb, (((1,), (1,)), ((), ())),
                                     preferred_element_type=jnp.float32)
            logg_ref[bb] = jnp.log(gb)
        c_ref[...] = jnp.zeros((N, N), jnp.float32)
        acc_ref[0] = 0.0
        acc_ref[1] = jnp.sum(g_ref[...] ** 2)

    eidx = edge_ref[...]
    a1 = _onehot(eidx[:, 0:1], BLK, N, jnp.bfloat16)
    a2 = _onehot(eidx[:, 1:2], BLK, N, jnp.bfloat16)
    c_ref[...] += jax.lax.dot_general(a1, a2, (((0,), (0,)), ((), ())),
                                      preferred_element_type=jnp.float32)

    tidx = tri_ref[...]
    at = (_onehot(tidx[:, 0:1], BLK, N, jnp.bfloat16)
          + _onehot(tidx[:, 1:2], BLK, N, jnp.bfloat16)
          + _onehot(tidx[:, 2:3], BLK, N, jnp.bfloat16))
    valid = tidx[:, 0:1] >= 0
    hs = jnp.dot(at, h_ref[...], preferred_element_type=jnp.float32)
    prod = jnp.float32(1.0)
    for bb in range(B):
        s = jnp.sum(jnp.exp(beta * hs[:, bb * K:(bb + 1) * K]),
                    axis=1, keepdims=True)
        prod = prod * s
    acc_ref[0] += jnp.sum(jnp.where(valid, jnp.log(prod), 0.0))

    @pl.when(i == nblocks - 1)
    def _fin():
        edge_lse = jnp.float32(0.0)
        for bb in range(B):
            edge_lse = edge_lse + jnp.sum(c_ref[...] * logg_ref[bb])
        total_lse = acc_ref[0] + edge_lse
        energy = (-(1.0 / (beta * num_simplices)) * total_lse
                  - 2.0 * acc_ref[1]) / (B * N)
        out_ref[...] = jnp.reshape(energy, (1, 1))


def kernel(g, patterns, beta, edges, triangles):
    B, N, D = g.shape
    K = patterns.shape[1]
    m2, m3 = edges.shape[0], triangles.shape[0]
    num_simplices = m2 + m3

    edges = edges.astype(jnp.int32)
    triangles = triangles.astype(jnp.int32)

    BLK = 1024
    nblocks = max((m2 + BLK - 1) // BLK, (m3 + BLK - 1) // BLK)
    edges = jnp.pad(edges, ((0, nblocks * BLK - m2), (0, 0)),
                    constant_values=-1)
    triangles = jnp.pad(triangles, ((0, nblocks * BLK - m3), (0, 0)),
                        constant_values=-1)
    beta_arr = jnp.reshape(beta.astype(jnp.float32), (1, 1))

    body = functools.partial(_body, B=B, N=N, K=K, BLK=BLK, nblocks=nblocks,
                             num_simplices=num_simplices)
    out = pl.pallas_call(
        body,
        grid=(nblocks,),
        in_specs=[
            pl.BlockSpec((BLK, 3), lambda i: (i, 0)),
            pl.BlockSpec((BLK, 2), lambda i: (i, 0)),
            pl.BlockSpec((B, N, D), lambda i: (0, 0, 0)),
            pl.BlockSpec((D, K), lambda i: (0, 0)),
            pl.BlockSpec((1, 1), lambda i: (0, 0)),
        ],
        out_specs=pl.BlockSpec((1, 1), lambda i: (0, 0)),
        out_shape=jax.ShapeDtypeStruct((1, 1), jnp.float32),
        scratch_shapes=[
            pltpu.VMEM((N, B * K), jnp.bfloat16),
            pltpu.VMEM((B, N, N), jnp.float32),
            pltpu.VMEM((N, N), jnp.float32),
            pltpu.SMEM((2,), jnp.float32),
        ],
    )(triangles, edges, g, patterns, beta_arr)
    return jnp.reshape(out, ())
